# baseline (speedup 1.0000x reference)
import functools

import numpy as np

import jax
import jax.numpy as jnp
from jax import lax
from jax.experimental import pallas as pl
from jax.experimental.pallas import tpu as pltpu
from jax.experimental.pallas import tpu_sc as plsc

_N = 1048576
_OHEM = 3
_NC, _NS, _L = 2, 16, 16
_NW = _NC * _NS
_PER_W = _N // _NW
_CH = 512
_NCH = _PER_W // _CH

_SP_C = (-1.7414117e-02, 8.2691424e-02, -1.9035463e-01, 3.1574753e-01,
         -4.9737328e-01, 9.9984771e-01, 1.4716139e-06)

@functools.cache
def _make_sc_reduce():
    mesh = plsc.VectorSubcoreMesh(core_axis_name="c", subcore_axis_name="s")
    return pl.kernel(
        _sc_reduce_body,
        out_type=jax.ShapeDtypeStruct((_NW, 2 * _L), jnp.float32),
        mesh=mesh,
        compiler_params=pltpu.CompilerParams(needs_layout_passes=False),
        scratch_types=[
            pltpu.VMEM((_CH, 2), jnp.float32),
            pltpu.VMEM((_CH, 2), jnp.float32),
            pltpu.VMEM((_PER_W,), jnp.int32),
            pltpu.VMEM((2 * _L,), jnp.float32),
            pltpu.SemaphoreType.DMA,
            pltpu.SemaphoreType.DMA,
        ],
    )


def _sc_reduce_body(pred_hbm, label_hbm, out_p, pbuf0, pbuf1, lbuf, obuf,
                    sem0, sem1):
    wid = lax.axis_index("s") * _NC + lax.axis_index("c")
    base = wid * _PER_W
    pltpu.sync_copy(label_hbm.at[pl.ds(base, _PER_W)], lbuf)

    iota = lax.iota(jnp.int32, _L)
    zi = jnp.zeros((_L,), jnp.int32)
    onei = jnp.full((_L,), 1, jnp.int32)
    zf = jnp.zeros((_L,), jnp.float32)
    onef = jnp.full((_L,), 1.0, jnp.float32)

    def chunk_acc(c, buf, accf, accc):
        for i in range(_CH // _L):
            rows = iota + i * _L
            g0 = plsc.load_gather(buf, [rows, zi])
            g1 = plsc.load_gather(buf, [rows, onei])
            lab = lbuf[pl.ds(c * _CH + i * _L, _L)]
            d = g1 - g0
            t = jnp.exp(-jnp.abs(d))
            sp = jnp.full((_L,), _SP_C[0], jnp.float32)
            for k in _SP_C[1:]:
                sp = sp * t + jnp.full((_L,), k, jnp.float32)
            isneg = lab == 0
            z = jnp.where(isneg, d, -d)
            accf = accf + jnp.maximum(z, zf) + sp
            accc = accc + jnp.where(isneg, zf, onef)
        return accf, accc

    def body(c, carry):
        accf, accc = carry
        pltpu.sync_copy(pred_hbm.at[pl.ds(base + c * _CH, _CH)], pbuf0)
        return chunk_acc(c, pbuf0, accf, accc)

    accf, accc = lax.fori_loop(
        0, _NCH, body,
        (jnp.zeros((_L,), jnp.float32), jnp.zeros((_L,), jnp.float32)))
    obuf[pl.ds(0, _L)] = accf
    obuf[pl.ds(_L, _L)] = accc
    pltpu.sync_copy(obuf, out_p.at[wid])


_RB = _N // 128
_NBLK = 16
_RPB = _RB // _NBLK

_MININT = -2147483648
_MAXPOS = 2147483647


def _skey(score):
    b = lax.bitcast_convert_type(score, jnp.int32)
    return jnp.where(b >= 0, b, b ^ jnp.int32(_MAXPOS))


def _rare_body(p0_ref, p1_ref, lab_ref, out_ref, si, sf):
    p = pl.program_id(0)
    b = pl.program_id(1)
    lab = lab_ref[...]
    neg = lab == 0

    @pl.when((p == 0) & (b == 0))
    def _():
        si[0] = 0

    @pl.when(p == 0)
    def _():
        si[0] = si[0] + jnp.sum((lab != 0).astype(jnp.int32))

    @pl.when((p >= 1) & (p <= 33) & (b == 0))
    def _():
        k = si[0] * _OHEM

        @pl.when(p == 1)
        def _():
            si[2] = 0

        @pl.when(p >= 2)
        def _():
            prevbit = jnp.left_shift(jnp.int32(1), 33 - p)
            si[2] = jnp.where(si[1] >= k, si[2] | prevbit, si[2])
        si[1] = 0

    @pl.when((p >= 1) & (p <= 32))
    def _():
        bit = jnp.left_shift(jnp.int32(1), 32 - p)
        scand = (si[2] | bit) ^ jnp.int32(_MININT)
        skey = _skey(p1_ref[...])
        si[1] = si[1] + jnp.sum((neg & (skey >= scand)).astype(jnp.int32))

    @pl.when((p == 33) & (b == 0))
    def _():
        k = si[0] * _OHEM
        ts = si[2] ^ jnp.int32(_MININT)
        si[4] = jnp.where(k == 0, jnp.int32(_MININT), ts)
        si[3] = 0
        sf[0] = 0.0

    @pl.when(p == 33)
    def _():
        p0 = p0_ref[...]
        p1 = p1_ref[...]
        skey = _skey(p1)
        m = (skey >= si[4]) | (lab != 0)
        mx = jnp.maximum(p0, p1)
        lse = mx + jnp.log(jnp.exp(p0 - mx) + jnp.exp(p1 - mx))
        nll = lse - jnp.where(lab == 0, p0, p1)
        sf[0] = sf[0] + jnp.sum(jnp.where(m, nll, 0.0))
        si[3] = si[3] + jnp.sum(m.astype(jnp.int32))

        @pl.when(b == _NBLK - 1)
        def _():
            out_ref[0] = sf[0] / jnp.maximum(si[3], 1).astype(jnp.float32)


def _rare(pred, label):
    p0 = pred[:, 0].reshape(_RB, 128)
    p1 = pred[:, 1].reshape(_RB, 128)
    lab = label.reshape(_RB, 128)
    out = pl.pallas_call(
        _rare_body,
        grid=(34, _NBLK),
        in_specs=[pl.BlockSpec((_RPB, 128), lambda p, b: (b, 0))] * 3,
        out_specs=pl.BlockSpec(memory_space=pltpu.MemorySpace.SMEM),
        out_shape=jax.ShapeDtypeStruct((1,), jnp.float32),
        scratch_shapes=[pltpu.SMEM((8,), jnp.int32),
                        pltpu.SMEM((4,), jnp.float32)],
    )(p0, p1, lab)
    return out[0]


def kernel(pred, label):
    parts = _make_sc_reduce()(pred, label)
    sums = jnp.sum(parts.reshape(_NW, 2, _L), axis=(0, 2))
    sum_nll = sums[0]
    pos_num = sums[1].astype(jnp.int32)
    n_neg = jnp.int32(_N) - pos_num
    return lax.cond(n_neg > pos_num * _OHEM,
                    lambda: _rare(pred, label),
                    lambda: sum_nll / jnp.float32(_N))


# speedup vs baseline: 2.2854x; 2.2854x over previous
import functools

import numpy as np

import jax
import jax.numpy as jnp
from jax import lax
from jax.experimental import pallas as pl
from jax.experimental.pallas import tpu as pltpu
from jax.experimental.pallas import tpu_sc as plsc

_N = 1048576
_OHEM = 3
_NC, _NS, _L = 2, 16, 16
_NW = _NC * _NS
_PER_W = _N // _NW
_ITERS = _PER_W // _L

_SP_C = (-1.7414117e-02, 8.2691424e-02, -1.9035463e-01, 3.1574753e-01,
         -4.9737328e-01, 9.9984771e-01, 1.4716139e-06)

@functools.cache
def _make_sc_reduce():
    mesh = plsc.VectorSubcoreMesh(core_axis_name="c", subcore_axis_name="s")
    return pl.kernel(
        _sc_reduce_body,
        out_type=jax.ShapeDtypeStruct((_NW, 2 * _L), jnp.float32),
        mesh=mesh,
        compiler_params=pltpu.CompilerParams(needs_layout_passes=False),
        scratch_types=[
            pltpu.VMEM((_PER_W,), jnp.float32),
            pltpu.VMEM((_PER_W,), jnp.float32),
            pltpu.VMEM((_PER_W,), jnp.int32),
            pltpu.VMEM((2 * _L,), jnp.float32),
        ],
    )


def _sc_reduce_body(p0_hbm, p1_hbm, label_hbm, out_p, p0buf, p1buf, lbuf, obuf):
    wid = lax.axis_index("s") * _NC + lax.axis_index("c")
    base = wid * _PER_W
    pltpu.sync_copy(p0_hbm.at[pl.ds(base, _PER_W)], p0buf)
    pltpu.sync_copy(p1_hbm.at[pl.ds(base, _PER_W)], p1buf)
    pltpu.sync_copy(label_hbm.at[pl.ds(base, _PER_W)], lbuf)

    zf = jnp.zeros((_L,), jnp.float32)
    onef = jnp.full((_L,), 1.0, jnp.float32)

    def body(i, carry):
        accf, accc = carry
        d = (p1buf[pl.ds(i * _L, _L)]
             - p0buf[pl.ds(i * _L, _L)])
        lab = lbuf[pl.ds(i * _L, _L)]
        t = jnp.exp(-jnp.abs(d))
        sp = jnp.full((_L,), _SP_C[0], jnp.float32)
        for c in _SP_C[1:]:
            sp = sp * t + jnp.full((_L,), c, jnp.float32)
        isneg = lab == 0
        z = jnp.where(isneg, d, -d)
        nll = jnp.maximum(z, zf) + sp
        return accf + nll, accc + jnp.where(isneg, zf, onef)

    accf, accc = lax.fori_loop(
        0, _ITERS, body,
        (jnp.zeros((_L,), jnp.float32), jnp.zeros((_L,), jnp.float32)))
    obuf[pl.ds(0, _L)] = accf
    obuf[pl.ds(_L, _L)] = accc
    pltpu.sync_copy(obuf, out_p.at[wid])


_RB = _N // 128
_NBLK = 16
_RPB = _RB // _NBLK

_MININT = -2147483648
_MAXPOS = 2147483647


def _skey(score):
    b = lax.bitcast_convert_type(score, jnp.int32)
    return jnp.where(b >= 0, b, b ^ jnp.int32(_MAXPOS))


def _rare_body(p0_ref, p1_ref, lab_ref, out_ref, si, sf):
    p = pl.program_id(0)
    b = pl.program_id(1)
    lab = lab_ref[...]
    neg = lab == 0

    @pl.when((p == 0) & (b == 0))
    def _():
        si[0] = 0

    @pl.when(p == 0)
    def _():
        si[0] = si[0] + jnp.sum((lab != 0).astype(jnp.int32))

    @pl.when((p >= 1) & (p <= 33) & (b == 0))
    def _():
        k = si[0] * _OHEM

        @pl.when(p == 1)
        def _():
            si[2] = 0

        @pl.when(p >= 2)
        def _():
            prevbit = jnp.left_shift(jnp.int32(1), 33 - p)
            si[2] = jnp.where(si[1] >= k, si[2] | prevbit, si[2])
        si[1] = 0

    @pl.when((p >= 1) & (p <= 32))
    def _():
        bit = jnp.left_shift(jnp.int32(1), 32 - p)
        scand = (si[2] | bit) ^ jnp.int32(_MININT)
        skey = _skey(p1_ref[...])
        si[1] = si[1] + jnp.sum((neg & (skey >= scand)).astype(jnp.int32))

    @pl.when((p == 33) & (b == 0))
    def _():
        k = si[0] * _OHEM
        ts = si[2] ^ jnp.int32(_MININT)
        si[4] = jnp.where(k == 0, jnp.int32(_MININT), ts)
        si[3] = 0
        sf[0] = 0.0

    @pl.when(p == 33)
    def _():
        p0 = p0_ref[...]
        p1 = p1_ref[...]
        skey = _skey(p1)
        m = (skey >= si[4]) | (lab != 0)
        mx = jnp.maximum(p0, p1)
        lse = mx + jnp.log(jnp.exp(p0 - mx) + jnp.exp(p1 - mx))
        nll = lse - jnp.where(lab == 0, p0, p1)
        sf[0] = sf[0] + jnp.sum(jnp.where(m, nll, 0.0))
        si[3] = si[3] + jnp.sum(m.astype(jnp.int32))

        @pl.when(b == _NBLK - 1)
        def _():
            out_ref[0] = sf[0] / jnp.maximum(si[3], 1).astype(jnp.float32)


def _rare(pred, label):
    p0 = pred[:, 0].reshape(_RB, 128)
    p1 = pred[:, 1].reshape(_RB, 128)
    lab = label.reshape(_RB, 128)
    out = pl.pallas_call(
        _rare_body,
        grid=(34, _NBLK),
        in_specs=[pl.BlockSpec((_RPB, 128), lambda p, b: (b, 0))] * 3,
        out_specs=pl.BlockSpec(memory_space=pltpu.MemorySpace.SMEM),
        out_shape=jax.ShapeDtypeStruct((1,), jnp.float32),
        scratch_shapes=[pltpu.SMEM((8,), jnp.int32),
                        pltpu.SMEM((4,), jnp.float32)],
    )(p0, p1, lab)
    return out[0]


def kernel(pred, label):
    idx0 = np.zeros((_N, 1), np.int32)
    idx1 = np.ones((_N, 1), np.int32)
    p0 = jnp.take_along_axis(pred, idx0, axis=1).reshape(_N)
    p1 = jnp.take_along_axis(pred, idx1, axis=1).reshape(_N)
    parts = _make_sc_reduce()(p0, p1, label)
    sums = jnp.sum(parts.reshape(_NW, 2, _L), axis=(0, 2))
    sum_nll = sums[0]
    pos_num = sums[1].astype(jnp.int32)
    n_neg = jnp.int32(_N) - pos_num
    return lax.cond(n_neg > pos_num * _OHEM,
                    lambda: _rare(pred, label),
                    lambda: sum_nll / jnp.float32(_N))


# speedup vs baseline: 4.3922x; 1.9219x over previous
import functools

import numpy as np

import jax
import jax.numpy as jnp
from jax import lax
from jax.experimental import pallas as pl
from jax.experimental.pallas import tpu as pltpu
from jax.experimental.pallas import tpu_sc as plsc

_N = 1048576
_OHEM = 3
_NC, _NS, _L = 2, 16, 16
_NW = _NC * _NS
_PER_W = _N // _NW
_ITERS = _PER_W // _L

_SP_C = (-1.7414117e-02, 8.2691424e-02, -1.9035463e-01, 3.1574753e-01,
         -4.9737328e-01, 9.9984771e-01, 1.4716139e-06)

@functools.cache
def _make_sc_reduce():
    mesh = plsc.VectorSubcoreMesh(core_axis_name="c", subcore_axis_name="s")
    return pl.kernel(
        _sc_reduce_body,
        out_type=jax.ShapeDtypeStruct((_NW, 2 * _L), jnp.float32),
        mesh=mesh,
        compiler_params=pltpu.CompilerParams(needs_layout_passes=False),
        scratch_types=[
            pltpu.VMEM((_PER_W,), jnp.float32),
            pltpu.VMEM((_PER_W,), jnp.float32),
            pltpu.VMEM((_PER_W,), jnp.int32),
            pltpu.VMEM((2 * _L,), jnp.float32),
        ],
    )


def _sc_reduce_body(p0_hbm, p1_hbm, label_hbm, out_p, p0buf, p1buf, lbuf, obuf):
    wid = lax.axis_index("s") * _NC + lax.axis_index("c")
    base = wid * _PER_W
    pltpu.sync_copy(p0_hbm.at[pl.ds(base, _PER_W)], p0buf)
    pltpu.sync_copy(p1_hbm.at[pl.ds(base, _PER_W)], p1buf)
    pltpu.sync_copy(label_hbm.at[pl.ds(base, _PER_W)], lbuf)

    zf = jnp.zeros((_L,), jnp.float32)
    onef = jnp.full((_L,), 1.0, jnp.float32)

    def body(i, carry):
        accf, accc = carry
        d = (p1buf[pl.ds(i * _L, _L)]
             - p0buf[pl.ds(i * _L, _L)])
        lab = lbuf[pl.ds(i * _L, _L)]
        t = jnp.exp(-jnp.abs(d))
        sp = jnp.full((_L,), _SP_C[0], jnp.float32)
        for c in _SP_C[1:]:
            sp = sp * t + jnp.full((_L,), c, jnp.float32)
        isneg = lab == 0
        z = jnp.where(isneg, d, -d)
        nll = jnp.maximum(z, zf) + sp
        return accf + nll, accc + jnp.where(isneg, zf, onef)

    accf, accc = lax.fori_loop(
        0, _ITERS, body,
        (jnp.zeros((_L,), jnp.float32), jnp.zeros((_L,), jnp.float32)))
    obuf[pl.ds(0, _L)] = accf
    obuf[pl.ds(_L, _L)] = accc
    pltpu.sync_copy(obuf, out_p.at[wid])


_RB = _N // 128
_NBLK = 16
_RPB = _RB // _NBLK

_MININT = -2147483648
_MAXPOS = 2147483647


def _skey(score):
    b = lax.bitcast_convert_type(score, jnp.int32)
    return jnp.where(b >= 0, b, b ^ jnp.int32(_MAXPOS))


def _rare_body(p0_ref, p1_ref, lab_ref, out_ref, si, sf):
    p = pl.program_id(0)
    b = pl.program_id(1)
    lab = lab_ref[...]
    neg = lab == 0

    @pl.when((p == 0) & (b == 0))
    def _():
        si[0] = 0

    @pl.when(p == 0)
    def _():
        si[0] = si[0] + jnp.sum((lab != 0).astype(jnp.int32))

    @pl.when((p >= 1) & (p <= 33) & (b == 0))
    def _():
        k = si[0] * _OHEM

        @pl.when(p == 1)
        def _():
            si[2] = 0

        @pl.when(p >= 2)
        def _():
            prevbit = jnp.left_shift(jnp.int32(1), 33 - p)
            si[2] = jnp.where(si[1] >= k, si[2] | prevbit, si[2])
        si[1] = 0

    @pl.when((p >= 1) & (p <= 32))
    def _():
        bit = jnp.left_shift(jnp.int32(1), 32 - p)
        scand = (si[2] | bit) ^ jnp.int32(_MININT)
        skey = _skey(p1_ref[...])
        si[1] = si[1] + jnp.sum((neg & (skey >= scand)).astype(jnp.int32))

    @pl.when((p == 33) & (b == 0))
    def _():
        k = si[0] * _OHEM
        ts = si[2] ^ jnp.int32(_MININT)
        si[4] = jnp.where(k == 0, jnp.int32(_MININT), ts)
        si[3] = 0
        sf[0] = 0.0

    @pl.when(p == 33)
    def _():
        p0 = p0_ref[...]
        p1 = p1_ref[...]
        skey = _skey(p1)
        m = (skey >= si[4]) | (lab != 0)
        mx = jnp.maximum(p0, p1)
        lse = mx + jnp.log(jnp.exp(p0 - mx) + jnp.exp(p1 - mx))
        nll = lse - jnp.where(lab == 0, p0, p1)
        sf[0] = sf[0] + jnp.sum(jnp.where(m, nll, 0.0))
        si[3] = si[3] + jnp.sum(m.astype(jnp.int32))

        @pl.when(b == _NBLK - 1)
        def _():
            out_ref[0] = sf[0] / jnp.maximum(si[3], 1).astype(jnp.float32)


def _rare(pred, label):
    p0 = pred[:, 0].reshape(_RB, 128)
    p1 = pred[:, 1].reshape(_RB, 128)
    lab = label.reshape(_RB, 128)
    out = pl.pallas_call(
        _rare_body,
        grid=(34, _NBLK),
        in_specs=[pl.BlockSpec((_RPB, 128), lambda p, b: (b, 0))] * 3,
        out_specs=pl.BlockSpec(memory_space=pltpu.MemorySpace.SMEM),
        out_shape=jax.ShapeDtypeStruct((1,), jnp.float32),
        scratch_shapes=[pltpu.SMEM((8,), jnp.int32),
                        pltpu.SMEM((4,), jnp.float32)],
    )(p0, p1, lab)
    return out[0]


def kernel(pred, label):
    idx1 = np.ones((_N, 1), np.int32)
    p0 = lax.slice(pred, (0, 0), (_N, 1)).reshape(_N)
    p1 = jnp.take_along_axis(pred, idx1, axis=1).reshape(_N)
    parts = _make_sc_reduce()(p0, p1, label)
    sums = jnp.sum(parts.reshape(_NW, 2, _L), axis=(0, 2))
    sum_nll = sums[0]
    pos_num = sums[1].astype(jnp.int32)
    n_neg = jnp.int32(_N) - pos_num
    return lax.cond(n_neg > pos_num * _OHEM,
                    lambda: _rare(pred, label),
                    lambda: sum_nll / jnp.float32(_N))


# speedup vs baseline: 10.3505x; 2.3566x over previous
import functools

import numpy as np

import jax
import jax.numpy as jnp
from jax import lax
from jax.experimental import pallas as pl
from jax.experimental.pallas import tpu as pltpu
from jax.experimental.pallas import tpu_sc as plsc

_N = 1048576
_OHEM = 3
_NC, _NS, _L = 2, 16, 16
_NW = _NC * _NS
_PER_W = _N // _NW
_ITERS = _PER_W // _L

_SP_C = (-1.7414117e-02, 8.2691424e-02, -1.9035463e-01, 3.1574753e-01,
         -4.9737328e-01, 9.9984771e-01, 1.4716139e-06)

@functools.cache
def _make_sc_reduce():
    mesh = plsc.VectorSubcoreMesh(core_axis_name="c", subcore_axis_name="s")
    return pl.kernel(
        _sc_reduce_body,
        out_type=jax.ShapeDtypeStruct((_NW, 2 * _L), jnp.float32),
        mesh=mesh,
        compiler_params=pltpu.CompilerParams(needs_layout_passes=False),
        scratch_types=[
            pltpu.VMEM((_PER_W,), jnp.float32),
            pltpu.VMEM((_PER_W,), jnp.float32),
            pltpu.VMEM((_PER_W,), jnp.int32),
            pltpu.VMEM((2 * _L,), jnp.float32),
        ],
    )


def _sc_reduce_body(p0_hbm, p1_hbm, label_hbm, out_p, p0buf, p1buf, lbuf, obuf):
    wid = lax.axis_index("s") * _NC + lax.axis_index("c")
    base = wid * _PER_W
    pltpu.sync_copy(p0_hbm.at[pl.ds(base, _PER_W)], p0buf)
    pltpu.sync_copy(p1_hbm.at[pl.ds(base, _PER_W)], p1buf)
    pltpu.sync_copy(label_hbm.at[pl.ds(base, _PER_W)], lbuf)

    zf = jnp.zeros((_L,), jnp.float32)
    onef = jnp.full((_L,), 1.0, jnp.float32)

    def body(i, carry):
        accf, accc = carry
        d = (p1buf[pl.ds(i * _L, _L)]
             - p0buf[pl.ds(i * _L, _L)])
        lab = lbuf[pl.ds(i * _L, _L)]
        t = jnp.exp(-jnp.abs(d))
        sp = jnp.full((_L,), _SP_C[0], jnp.float32)
        for c in _SP_C[1:]:
            sp = sp * t + jnp.full((_L,), c, jnp.float32)
        isneg = lab == 0
        z = jnp.where(isneg, d, -d)
        nll = jnp.maximum(z, zf) + sp
        return accf + nll, accc + jnp.where(isneg, zf, onef)

    accf, accc = lax.fori_loop(
        0, _ITERS, body,
        (jnp.zeros((_L,), jnp.float32), jnp.zeros((_L,), jnp.float32)))
    obuf[pl.ds(0, _L)] = accf
    obuf[pl.ds(_L, _L)] = accc
    pltpu.sync_copy(obuf, out_p.at[wid])


_RB = _N // 128
_NBLK = 16
_RPB = _RB // _NBLK

_MININT = -2147483648
_MAXPOS = 2147483647


def _skey(score):
    b = lax.bitcast_convert_type(score, jnp.int32)
    return jnp.where(b >= 0, b, b ^ jnp.int32(_MAXPOS))


def _rare_body(p0_ref, p1_ref, lab_ref, out_ref, si, sf):
    p = pl.program_id(0)
    b = pl.program_id(1)
    lab = lab_ref[...]
    neg = lab == 0

    @pl.when((p == 0) & (b == 0))
    def _():
        si[0] = 0

    @pl.when(p == 0)
    def _():
        si[0] = si[0] + jnp.sum((lab != 0).astype(jnp.int32))

    @pl.when((p >= 1) & (p <= 33) & (b == 0))
    def _():
        k = si[0] * _OHEM

        @pl.when(p == 1)
        def _():
            si[2] = 0

        @pl.when(p >= 2)
        def _():
            prevbit = jnp.left_shift(jnp.int32(1), 33 - p)
            si[2] = jnp.where(si[1] >= k, si[2] | prevbit, si[2])
        si[1] = 0

    @pl.when((p >= 1) & (p <= 32))
    def _():
        bit = jnp.left_shift(jnp.int32(1), 32 - p)
        scand = (si[2] | bit) ^ jnp.int32(_MININT)
        skey = _skey(p1_ref[...])
        si[1] = si[1] + jnp.sum((neg & (skey >= scand)).astype(jnp.int32))

    @pl.when((p == 33) & (b == 0))
    def _():
        k = si[0] * _OHEM
        ts = si[2] ^ jnp.int32(_MININT)
        si[4] = jnp.where(k == 0, jnp.int32(_MININT), ts)
        si[3] = 0
        sf[0] = 0.0

    @pl.when(p == 33)
    def _():
        p0 = p0_ref[...]
        p1 = p1_ref[...]
        skey = _skey(p1)
        m = (skey >= si[4]) | (lab != 0)
        mx = jnp.maximum(p0, p1)
        lse = mx + jnp.log(jnp.exp(p0 - mx) + jnp.exp(p1 - mx))
        nll = lse - jnp.where(lab == 0, p0, p1)
        sf[0] = sf[0] + jnp.sum(jnp.where(m, nll, 0.0))
        si[3] = si[3] + jnp.sum(m.astype(jnp.int32))

        @pl.when(b == _NBLK - 1)
        def _():
            out_ref[0] = sf[0] / jnp.maximum(si[3], 1).astype(jnp.float32)


def _rare(pred, label):
    p0 = pred[:, 0].reshape(_RB, 128)
    p1 = pred[:, 1].reshape(_RB, 128)
    lab = label.reshape(_RB, 128)
    out = pl.pallas_call(
        _rare_body,
        grid=(34, _NBLK),
        in_specs=[pl.BlockSpec((_RPB, 128), lambda p, b: (b, 0))] * 3,
        out_specs=pl.BlockSpec(memory_space=pltpu.MemorySpace.SMEM),
        out_shape=jax.ShapeDtypeStruct((1,), jnp.float32),
        scratch_shapes=[pltpu.SMEM((8,), jnp.int32),
                        pltpu.SMEM((4,), jnp.float32)],
    )(p0, p1, lab)
    return out[0]


def kernel(pred, label):
    p0 = lax.slice(pred, (0, 0), (_N, 1)).reshape(_N)
    p1 = lax.slice(pred, (0, 1), (_N, 2)).reshape(_N)
    parts = _make_sc_reduce()(p0, p1, label)
    sums = jnp.sum(parts.reshape(_NW, 2, _L), axis=(0, 2))
    sum_nll = sums[0]
    pos_num = sums[1].astype(jnp.int32)
    n_neg = jnp.int32(_N) - pos_num
    return lax.cond(n_neg > pos_num * _OHEM,
                    lambda: _rare(pred, label),
                    lambda: sum_nll / jnp.float32(_N))


# speedup vs baseline: 11.0375x; 1.0664x over previous
import functools

import numpy as np

import jax
import jax.numpy as jnp
from jax import lax
from jax.experimental import pallas as pl
from jax.experimental.pallas import tpu as pltpu
from jax.experimental.pallas import tpu_sc as plsc

_N = 1048576
_OHEM = 3
_NC, _NS, _L = 2, 16, 16
_NW = _NC * _NS
_PER_W = _N // _NW
_CH = 8192
_NCH = _PER_W // _CH

_SP_C = (-1.7414117e-02, 8.2691424e-02, -1.9035463e-01, 3.1574753e-01,
         -4.9737328e-01, 9.9984771e-01, 1.4716139e-06)

@functools.cache
def _make_sc_reduce():
    mesh = plsc.VectorSubcoreMesh(core_axis_name="c", subcore_axis_name="s")
    return pl.kernel(
        _sc_reduce_body,
        out_type=jax.ShapeDtypeStruct((_NW, 2 * _L), jnp.float32),
        mesh=mesh,
        compiler_params=pltpu.CompilerParams(needs_layout_passes=False),
        scratch_types=[
            pltpu.VMEM((_CH,), jnp.float32),
            pltpu.VMEM((_CH,), jnp.float32),
            pltpu.VMEM((_CH,), jnp.float32),
            pltpu.VMEM((_CH,), jnp.float32),
            pltpu.VMEM((_CH,), jnp.int32),
            pltpu.VMEM((_CH,), jnp.int32),
            pltpu.VMEM((2 * _L,), jnp.float32),
            pltpu.SemaphoreType.DMA,
            pltpu.SemaphoreType.DMA,
        ],
    )


def _sc_reduce_body(p0_hbm, p1_hbm, label_hbm, out_p,
                    p0a, p0b, p1a, p1b, laba, labb, obuf, sema, semb):
    wid = lax.axis_index("s") * _NC + lax.axis_index("c")
    base = wid * _PER_W
    bufs = ((p0a, p1a, laba, sema), (p0b, p1b, labb, semb))

    def start(c, bp0, bp1, bl, sem):
        off = base + c * _CH
        pltpu.async_copy(p0_hbm.at[pl.ds(off, _CH)], bp0, sem)
        pltpu.async_copy(p1_hbm.at[pl.ds(off, _CH)], bp1, sem)
        pltpu.async_copy(label_hbm.at[pl.ds(off, _CH)], bl, sem)

    def drain(c, bp0, bp1, bl, sem):
        off = base + c * _CH
        pltpu.make_async_copy(p0_hbm.at[pl.ds(off, _CH)], bp0, sem).wait()
        pltpu.make_async_copy(p1_hbm.at[pl.ds(off, _CH)], bp1, sem).wait()
        pltpu.make_async_copy(label_hbm.at[pl.ds(off, _CH)], bl, sem).wait()

    zf = jnp.zeros((_L,), jnp.float32)
    onef = jnp.full((_L,), 1.0, jnp.float32)

    def chunk(bp0, bp1, bl, carry):
        def body(i, cr):
            accf, accc = cr
            d = bp1[pl.ds(i * _L, _L)] - bp0[pl.ds(i * _L, _L)]
            lab = bl[pl.ds(i * _L, _L)]
            t = jnp.exp(-jnp.abs(d))
            sp = jnp.full((_L,), _SP_C[0], jnp.float32)
            for c in _SP_C[1:]:
                sp = sp * t + jnp.full((_L,), c, jnp.float32)
            isneg = lab == 0
            z = jnp.where(isneg, d, -d)
            nll = jnp.maximum(z, zf) + sp
            return accf + nll, accc + jnp.where(isneg, zf, onef)
        return lax.fori_loop(0, _CH // _L, body, carry)

    carry = (jnp.zeros((_L,), jnp.float32), jnp.zeros((_L,), jnp.float32))
    start(0, *bufs[0])
    for c in range(_NCH):
        if c + 1 < _NCH:
            start(c + 1, *bufs[(c + 1) % 2])
        drain(c, *bufs[c % 2])
        carry = chunk(*bufs[c % 2][:3], carry)
    accf, accc = carry
    obuf[pl.ds(0, _L)] = accf
    obuf[pl.ds(_L, _L)] = accc
    pltpu.sync_copy(obuf, out_p.at[wid])


_RB = _N // 128
_NBLK = 16
_RPB = _RB // _NBLK

_MININT = -2147483648
_MAXPOS = 2147483647


def _skey(score):
    b = lax.bitcast_convert_type(score, jnp.int32)
    return jnp.where(b >= 0, b, b ^ jnp.int32(_MAXPOS))


def _rare_body(p0_ref, p1_ref, lab_ref, out_ref, si, sf):
    p = pl.program_id(0)
    b = pl.program_id(1)
    lab = lab_ref[...]
    neg = lab == 0

    @pl.when((p == 0) & (b == 0))
    def _():
        si[0] = 0

    @pl.when(p == 0)
    def _():
        si[0] = si[0] + jnp.sum((lab != 0).astype(jnp.int32))

    @pl.when((p >= 1) & (p <= 33) & (b == 0))
    def _():
        k = si[0] * _OHEM

        @pl.when(p == 1)
        def _():
            si[2] = 0

        @pl.when(p >= 2)
        def _():
            prevbit = jnp.left_shift(jnp.int32(1), 33 - p)
            si[2] = jnp.where(si[1] >= k, si[2] | prevbit, si[2])
        si[1] = 0

    @pl.when((p >= 1) & (p <= 32))
    def _():
        bit = jnp.left_shift(jnp.int32(1), 32 - p)
        scand = (si[2] | bit) ^ jnp.int32(_MININT)
        skey = _skey(p1_ref[...])
        si[1] = si[1] + jnp.sum((neg & (skey >= scand)).astype(jnp.int32))

    @pl.when((p == 33) & (b == 0))
    def _():
        k = si[0] * _OHEM
        ts = si[2] ^ jnp.int32(_MININT)
        si[4] = jnp.where(k == 0, jnp.int32(_MININT), ts)
        si[3] = 0
        sf[0] = 0.0

    @pl.when(p == 33)
    def _():
        p0 = p0_ref[...]
        p1 = p1_ref[...]
        skey = _skey(p1)
        m = (skey >= si[4]) | (lab != 0)
        mx = jnp.maximum(p0, p1)
        lse = mx + jnp.log(jnp.exp(p0 - mx) + jnp.exp(p1 - mx))
        nll = lse - jnp.where(lab == 0, p0, p1)
        sf[0] = sf[0] + jnp.sum(jnp.where(m, nll, 0.0))
        si[3] = si[3] + jnp.sum(m.astype(jnp.int32))

        @pl.when(b == _NBLK - 1)
        def _():
            out_ref[0] = sf[0] / jnp.maximum(si[3], 1).astype(jnp.float32)


def _rare(pred, label):
    p0 = pred[:, 0].reshape(_RB, 128)
    p1 = pred[:, 1].reshape(_RB, 128)
    lab = label.reshape(_RB, 128)
    out = pl.pallas_call(
        _rare_body,
        grid=(34, _NBLK),
        in_specs=[pl.BlockSpec((_RPB, 128), lambda p, b: (b, 0))] * 3,
        out_specs=pl.BlockSpec(memory_space=pltpu.MemorySpace.SMEM),
        out_shape=jax.ShapeDtypeStruct((1,), jnp.float32),
        scratch_shapes=[pltpu.SMEM((8,), jnp.int32),
                        pltpu.SMEM((4,), jnp.float32)],
    )(p0, p1, lab)
    return out[0]


def kernel(pred, label):
    p0 = lax.slice(pred, (0, 0), (_N, 1)).reshape(_N)
    p1 = lax.slice(pred, (0, 1), (_N, 2)).reshape(_N)
    parts = _make_sc_reduce()(p0, p1, label)
    sums = jnp.sum(parts.reshape(_NW, 2, _L), axis=(0, 2))
    sum_nll = sums[0]
    pos_num = sums[1].astype(jnp.int32)
    n_neg = jnp.int32(_N) - pos_num
    return lax.cond(n_neg > pos_num * _OHEM,
                    lambda: _rare(pred, label),
                    lambda: sum_nll / jnp.float32(_N))
